# baseline (speedup 1.0000x reference)
import functools

import jax
import jax.numpy as jnp
from jax import lax
from jax.experimental import pallas as pl
from jax.experimental.pallas import tpu as pltpu
from jax.experimental.pallas import tpu_sc as plsc

N = 10000
D = 128
E = 320000

NUM_CORES = 2
NUM_SUBCORES = 16
NUM_WORKERS = NUM_CORES * NUM_SUBCORES

NPAD = 10240
ROWS_PER_TILE = NPAD // NUM_SUBCORES
CHUNK = 128
IDX_ROWS = 80
EPAD = NUM_WORKERS * IDX_ROWS * CHUNK
BLK = 2000
GRID = N // BLK


def _agg_body(x_hbm, src_hbm, dst_hbm, out_hbm, src_v, dst_v, buf, agg_sh,
              sem):
    c = lax.axis_index("c")
    s = lax.axis_index("s")
    wid = s * NUM_CORES + c

    zeros16 = jnp.zeros((16,), jnp.float32)

    def zrow(i, carry):
        for j in range(8):
            buf[i, pl.ds(j * 16, 16)] = zeros16
        return carry

    lax.fori_loop(0, CHUNK, zrow, 0)
    for k in range(ROWS_PER_TILE // CHUNK):
        pltpu.sync_copy(buf,
                        agg_sh.at[pl.ds(s * ROWS_PER_TILE + k * CHUNK, CHUNK)])
    plsc.subcore_barrier()

    pltpu.sync_copy(src_hbm.at[pl.ds(wid * IDX_ROWS, IDX_ROWS)], src_v)
    pltpu.sync_copy(dst_hbm.at[pl.ds(wid * IDX_ROWS, IDX_ROWS)], dst_v)

    def ebody(j, carry):
        pltpu.async_copy(x_hbm.at[src_v.at[j]], buf, sem).wait()
        pltpu.sync_copy(buf, agg_sh.at[dst_v.at[j]], add=True)
        return carry

    lax.fori_loop(0, IDX_ROWS, ebody, 0)

    plsc.subcore_barrier()
    pltpu.sync_copy(agg_sh.at[pl.ds(s * ROWS_PER_TILE, ROWS_PER_TILE)],
                    out_hbm.at[c, pl.ds(s * ROWS_PER_TILE, ROWS_PER_TILE)])


@jax.jit
def _sc_agg(xf, src2d, dst2d):
    mesh = plsc.VectorSubcoreMesh(core_axis_name="c", subcore_axis_name="s")
    kern = pl.kernel(
        _agg_body,
        out_type=jax.ShapeDtypeStruct((NUM_CORES, NPAD, D), jnp.float32),
        mesh=mesh,
        scratch_types=[
            pltpu.VMEM((IDX_ROWS, CHUNK), jnp.int32),
            pltpu.VMEM((IDX_ROWS, CHUNK), jnp.int32),
            pltpu.VMEM((CHUNK, D), jnp.float32),
            pltpu.VMEM_SHARED((NPAD, D), jnp.float32),
            pltpu.SemaphoreType.DMA,
        ],
    )
    return kern(xf, src2d, dst2d)


def _mlp1_body(x_ref, slab_ref, w_ref, b_ref, t_ref, stats_ref):
    y = x_ref[...] + slab_ref[0] + slab_ref[1]
    t = jnp.dot(y, w_ref[...], preferred_element_type=jnp.float32) + b_ref[...]
    t_ref[...] = t

    @pl.when(pl.program_id(0) == 0)
    def _():
        stats_ref[...] = jnp.zeros_like(stats_ref)

    stats_ref[0:1, :] += jnp.sum(t, axis=0, keepdims=True)
    stats_ref[1:2, :] += jnp.sum(t * t, axis=0, keepdims=True)


def _mlp1(x, slab, w1, b1):
    return pl.pallas_call(
        _mlp1_body,
        grid=(GRID,),
        in_specs=[
            pl.BlockSpec((BLK, D), lambda i: (i, 0)),
            pl.BlockSpec((NUM_CORES, BLK, D), lambda i: (0, i, 0)),
            pl.BlockSpec((D, D), lambda i: (0, 0)),
            pl.BlockSpec((1, D), lambda i: (0, 0)),
        ],
        out_specs=[
            pl.BlockSpec((BLK, D), lambda i: (i, 0)),
            pl.BlockSpec((2, D), lambda i: (0, 0)),
        ],
        out_shape=[
            jax.ShapeDtypeStruct((N, D), jnp.float32),
            jax.ShapeDtypeStruct((2, D), jnp.float32),
        ],
    )(x, slab, w1, b1.reshape(1, D))


def _bn_head(t, stats, s_ref, be_ref):
    mu = stats[0:1, :] * (1.0 / N)
    var = stats[1:2, :] * (1.0 / N) - mu * mu
    z = (t - mu) * lax.rsqrt(var + 1e-5) * s_ref[...] + be_ref[...]
    return jnp.maximum(z, 0.0)


def _mlp2_body(t_ref, stats_ref, s_ref, be_ref, w_ref, b_ref, h_ref):
    z = _bn_head(t_ref[...], stats_ref[...], s_ref, be_ref)
    h = jnp.dot(z, w_ref[...], preferred_element_type=jnp.float32) + b_ref[...]
    h_ref[...] = jnp.maximum(h, 0.0)


def _mlp2(t, stats, s, be, w2, b2):
    return pl.pallas_call(
        _mlp2_body,
        grid=(GRID,),
        in_specs=[
            pl.BlockSpec((BLK, D), lambda i: (i, 0)),
            pl.BlockSpec((2, D), lambda i: (0, 0)),
            pl.BlockSpec((1, D), lambda i: (0, 0)),
            pl.BlockSpec((1, D), lambda i: (0, 0)),
            pl.BlockSpec((D, D), lambda i: (0, 0)),
            pl.BlockSpec((1, D), lambda i: (0, 0)),
        ],
        out_specs=pl.BlockSpec((BLK, D), lambda i: (i, 0)),
        out_shape=jax.ShapeDtypeStruct((N, D), jnp.float32),
    )(t, stats, s.reshape(1, D), be.reshape(1, D), w2, b2.reshape(1, D))


def _mlp2f_body(t_ref, stats_ref, s_ref, be_ref, w_ref, b_ref,
                mw1_ref, mb1_ref, mw2_ref, mb2_ref, o_ref):
    z = _bn_head(t_ref[...], stats_ref[...], s_ref, be_ref)
    h = jnp.dot(z, w_ref[...], preferred_element_type=jnp.float32) + b_ref[...]
    h = jnp.maximum(h, 0.0)
    g = jnp.dot(h, mw1_ref[...], preferred_element_type=jnp.float32) + mb1_ref[...]
    g = jnp.maximum(g, 0.0)
    o_ref[...] = jnp.dot(g, mw2_ref[...], preferred_element_type=jnp.float32) + mb2_ref[...]


def _mlp2f(t, stats, s, be, w2, b2, mw1, mb1, mw2, mb2):
    Co = mw2.shape[1]
    return pl.pallas_call(
        _mlp2f_body,
        grid=(GRID,),
        in_specs=[
            pl.BlockSpec((BLK, D), lambda i: (i, 0)),
            pl.BlockSpec((2, D), lambda i: (0, 0)),
            pl.BlockSpec((1, D), lambda i: (0, 0)),
            pl.BlockSpec((1, D), lambda i: (0, 0)),
            pl.BlockSpec((D, D), lambda i: (0, 0)),
            pl.BlockSpec((1, D), lambda i: (0, 0)),
            pl.BlockSpec((D, D), lambda i: (0, 0)),
            pl.BlockSpec((1, D), lambda i: (0, 0)),
            pl.BlockSpec((D, Co), lambda i: (0, 0)),
            pl.BlockSpec((1, Co), lambda i: (0, 0)),
        ],
        out_specs=pl.BlockSpec((BLK, Co), lambda i: (i, 0)),
        out_shape=jax.ShapeDtypeStruct((N, Co), jnp.float32),
    )(t, stats, s.reshape(1, D), be.reshape(1, D), w2, b2.reshape(1, D),
      mw1, mb1.reshape(1, D), mw2, mb2.reshape(1, Co))


def kernel(x, edge_index, c1_W1, c1_b1, c1_s, c1_be, c1_W2, c1_b2,
           c2_W1, c2_b1, c2_s, c2_be, c2_W2, c2_b2,
           m_W1, m_b1, m_W2, m_b2):
    ei = edge_index.astype(jnp.int32)
    pad = EPAD - E
    src2d = jnp.concatenate(
        [ei[0], jnp.zeros((pad,), jnp.int32)]).reshape(-1, CHUNK)
    dst2d = jnp.concatenate(
        [ei[1], jnp.full((pad,), N, jnp.int32)]).reshape(-1, CHUNK)

    slab1 = _sc_agg(x, src2d, dst2d)
    t1, stats1 = _mlp1(x, slab1, c1_W1, c1_b1)
    h1 = _mlp2(t1, stats1, c1_s, c1_be, c1_W2, c1_b2)

    slab2 = _sc_agg(h1, src2d, dst2d)
    t2, stats2 = _mlp1(h1, slab2, c2_W1, c2_b1)
    return _mlp2f(t2, stats2, c2_s, c2_be, c2_W2, c2_b2,
                  m_W1, m_b1, m_W2, m_b2)


# speedup vs baseline: 1.0026x; 1.0026x over previous
import functools

import jax
import jax.numpy as jnp
from jax import lax
from jax.experimental import pallas as pl
from jax.experimental.pallas import tpu as pltpu
from jax.experimental.pallas import tpu_sc as plsc

N = 10000
D = 128
E = 320000

NUM_CORES = 2
NUM_SUBCORES = 16
NUM_WORKERS = NUM_CORES * NUM_SUBCORES

NPAD = 10240
ROWS_PER_TILE = NPAD // NUM_SUBCORES
CHUNK = 128
IDX_ROWS = 80
EPAD = NUM_WORKERS * IDX_ROWS * CHUNK
BLK = 2000
GRID = N // BLK


def _agg_body(x_hbm, src_hbm, dst_hbm, out_hbm, src_v, dst_v, buf, agg_sh,
              sem):
    c = lax.axis_index("c")
    s = lax.axis_index("s")
    wid = c * NUM_SUBCORES + s

    zeros16 = jnp.zeros((16,), jnp.float32)

    def zrow(i, carry):
        for j in range(8):
            buf[i, pl.ds(j * 16, 16)] = zeros16
        return carry

    lax.fori_loop(0, CHUNK, zrow, 0)
    for k in range(ROWS_PER_TILE // CHUNK):
        pltpu.sync_copy(buf,
                        agg_sh.at[pl.ds(s * ROWS_PER_TILE + k * CHUNK, CHUNK)])
    plsc.subcore_barrier()

    pltpu.sync_copy(src_hbm.at[pl.ds(wid * IDX_ROWS, IDX_ROWS)], src_v)
    pltpu.sync_copy(dst_hbm.at[pl.ds(wid * IDX_ROWS, IDX_ROWS)], dst_v)

    def ebody(j, carry):
        pltpu.async_copy(x_hbm.at[src_v.at[j]], buf, sem).wait()
        pltpu.sync_copy(buf, agg_sh.at[dst_v.at[j]], add=True)
        return carry

    lax.fori_loop(0, IDX_ROWS, ebody, 0)

    plsc.subcore_barrier()
    pltpu.sync_copy(agg_sh.at[pl.ds(s * ROWS_PER_TILE, ROWS_PER_TILE)],
                    out_hbm.at[c, pl.ds(s * ROWS_PER_TILE, ROWS_PER_TILE)])


@jax.jit
def _sc_agg(xf, src2d, dst2d):
    mesh = plsc.VectorSubcoreMesh(core_axis_name="c", subcore_axis_name="s")
    kern = pl.kernel(
        _agg_body,
        out_type=jax.ShapeDtypeStruct((NUM_CORES, NPAD, D), jnp.float32),
        mesh=mesh,
        scratch_types=[
            pltpu.VMEM((IDX_ROWS, CHUNK), jnp.int32),
            pltpu.VMEM((IDX_ROWS, CHUNK), jnp.int32),
            pltpu.VMEM((CHUNK, D), jnp.float32),
            pltpu.VMEM_SHARED((NPAD, D), jnp.float32),
            pltpu.SemaphoreType.DMA,
        ],
    )
    return kern(xf, src2d, dst2d)


def _mlp1_body(x_ref, slab_ref, w_ref, b_ref, t_ref, stats_ref):
    y = x_ref[...] + slab_ref[0] + slab_ref[1]
    t = jnp.dot(y, w_ref[...], preferred_element_type=jnp.float32) + b_ref[...]
    t_ref[...] = t

    @pl.when(pl.program_id(0) == 0)
    def _():
        stats_ref[...] = jnp.zeros_like(stats_ref)

    stats_ref[0:1, :] += jnp.sum(t, axis=0, keepdims=True)
    stats_ref[1:2, :] += jnp.sum(t * t, axis=0, keepdims=True)


def _mlp1(x, slab, w1, b1):
    return pl.pallas_call(
        _mlp1_body,
        grid=(GRID,),
        in_specs=[
            pl.BlockSpec((BLK, D), lambda i: (i, 0)),
            pl.BlockSpec((NUM_CORES, BLK, D), lambda i: (0, i, 0)),
            pl.BlockSpec((D, D), lambda i: (0, 0)),
            pl.BlockSpec((1, D), lambda i: (0, 0)),
        ],
        out_specs=[
            pl.BlockSpec((BLK, D), lambda i: (i, 0)),
            pl.BlockSpec((2, D), lambda i: (0, 0)),
        ],
        out_shape=[
            jax.ShapeDtypeStruct((N, D), jnp.float32),
            jax.ShapeDtypeStruct((2, D), jnp.float32),
        ],
    )(x, slab, w1, b1.reshape(1, D))


def _bn_head(t, stats, s_ref, be_ref):
    mu = stats[0:1, :] * (1.0 / N)
    var = stats[1:2, :] * (1.0 / N) - mu * mu
    z = (t - mu) * lax.rsqrt(var + 1e-5) * s_ref[...] + be_ref[...]
    return jnp.maximum(z, 0.0)


def _mlp2_body(t_ref, stats_ref, s_ref, be_ref, w_ref, b_ref, h_ref):
    z = _bn_head(t_ref[...], stats_ref[...], s_ref, be_ref)
    h = jnp.dot(z, w_ref[...], preferred_element_type=jnp.float32) + b_ref[...]
    h_ref[...] = jnp.maximum(h, 0.0)


def _mlp2(t, stats, s, be, w2, b2):
    return pl.pallas_call(
        _mlp2_body,
        grid=(GRID,),
        in_specs=[
            pl.BlockSpec((BLK, D), lambda i: (i, 0)),
            pl.BlockSpec((2, D), lambda i: (0, 0)),
            pl.BlockSpec((1, D), lambda i: (0, 0)),
            pl.BlockSpec((1, D), lambda i: (0, 0)),
            pl.BlockSpec((D, D), lambda i: (0, 0)),
            pl.BlockSpec((1, D), lambda i: (0, 0)),
        ],
        out_specs=pl.BlockSpec((BLK, D), lambda i: (i, 0)),
        out_shape=jax.ShapeDtypeStruct((N, D), jnp.float32),
    )(t, stats, s.reshape(1, D), be.reshape(1, D), w2, b2.reshape(1, D))


def _mlp2f_body(t_ref, stats_ref, s_ref, be_ref, w_ref, b_ref,
                mw1_ref, mb1_ref, mw2_ref, mb2_ref, o_ref):
    z = _bn_head(t_ref[...], stats_ref[...], s_ref, be_ref)
    h = jnp.dot(z, w_ref[...], preferred_element_type=jnp.float32) + b_ref[...]
    h = jnp.maximum(h, 0.0)
    g = jnp.dot(h, mw1_ref[...], preferred_element_type=jnp.float32) + mb1_ref[...]
    g = jnp.maximum(g, 0.0)
    o_ref[...] = jnp.dot(g, mw2_ref[...], preferred_element_type=jnp.float32) + mb2_ref[...]


def _mlp2f(t, stats, s, be, w2, b2, mw1, mb1, mw2, mb2):
    Co = mw2.shape[1]
    return pl.pallas_call(
        _mlp2f_body,
        grid=(GRID,),
        in_specs=[
            pl.BlockSpec((BLK, D), lambda i: (i, 0)),
            pl.BlockSpec((2, D), lambda i: (0, 0)),
            pl.BlockSpec((1, D), lambda i: (0, 0)),
            pl.BlockSpec((1, D), lambda i: (0, 0)),
            pl.BlockSpec((D, D), lambda i: (0, 0)),
            pl.BlockSpec((1, D), lambda i: (0, 0)),
            pl.BlockSpec((D, D), lambda i: (0, 0)),
            pl.BlockSpec((1, D), lambda i: (0, 0)),
            pl.BlockSpec((D, Co), lambda i: (0, 0)),
            pl.BlockSpec((1, Co), lambda i: (0, 0)),
        ],
        out_specs=pl.BlockSpec((BLK, Co), lambda i: (i, 0)),
        out_shape=jax.ShapeDtypeStruct((N, Co), jnp.float32),
    )(t, stats, s.reshape(1, D), be.reshape(1, D), w2, b2.reshape(1, D),
      mw1, mb1.reshape(1, D), mw2, mb2.reshape(1, Co))


def kernel(x, edge_index, c1_W1, c1_b1, c1_s, c1_be, c1_W2, c1_b2,
           c2_W1, c2_b1, c2_s, c2_be, c2_W2, c2_b2,
           m_W1, m_b1, m_W2, m_b2):
    ei = edge_index.astype(jnp.int32)
    pad = EPAD - E
    src2d = jnp.concatenate(
        [ei[0], jnp.zeros((pad,), jnp.int32)]).reshape(-1, CHUNK)
    dst2d = jnp.concatenate(
        [ei[1], jnp.full((pad,), N, jnp.int32)]).reshape(-1, CHUNK)

    slab1 = _sc_agg(x, src2d, dst2d)
    t1, stats1 = _mlp1(x, slab1, c1_W1, c1_b1)
    h1 = _mlp2(t1, stats1, c1_s, c1_be, c1_W2, c1_b2)

    slab2 = _sc_agg(h1, src2d, dst2d)
    t2, stats2 = _mlp1(h1, slab2, c2_W1, c2_b1)
    return _mlp2f(t2, stats2, c2_s, c2_be, c2_W2, c2_b2,
                  m_W1, m_b1, m_W2, m_b2)
